# baseline (speedup 1.0000x reference)
import functools

import jax
import jax.numpy as jnp
from jax import lax
from jax.experimental import pallas as pl
from jax.experimental.pallas import tpu as pltpu, tpu_sc as plsc

B, S = 128, 512
V, H, P, T = 30522, 768, 512, 2
EPS = 1e-12

NW = 32
C = 64
NCHUNK_S = S // C
BG = B // (NW // NCHUNK_S)
NV = H // 16


def _body(ids_hbm, tt_hbm, word_hbm, pos_hbm, typ_hbm, gamma_hbm, beta_hbm,
          out_hbm, idx_v, tt_v, pos_v, rows_v, typ_v, tdiff_v, gamma_v,
          beta_v, sem):
    wid = lax.axis_index("s") * 2 + lax.axis_index("c")
    c = lax.rem(wid, NCHUNK_S)
    bg = lax.div(wid, NCHUNK_S)
    s0 = c * C

    pltpu.sync_copy(pos_hbm.at[pl.ds(s0, C)], pos_v)
    pltpu.sync_copy(typ_hbm, typ_v)
    pltpu.sync_copy(gamma_hbm, gamma_v)
    pltpu.sync_copy(beta_hbm, beta_v)

    for j in range(NV):
        sl = pl.ds(j * 16, 16)
        tdiff_v[sl] = typ_v[1, sl] - typ_v[0, sl]

    def prep(t, carry):
        for j in range(NV):
            sl = pl.ds(j * 16, 16)
            pos_v[t, sl] = pos_v[t, sl] + typ_v[0, sl]
        return carry

    lax.fori_loop(0, C, prep, 0)

    def chunk_body(i, carry):
        base = (bg * BG + i) * S + s0
        pltpu.sync_copy(ids_hbm.at[pl.ds(base, C)], idx_v)
        pltpu.sync_copy(tt_hbm.at[pl.ds(base, C)], tt_v)
        pltpu.async_copy(word_hbm.at[idx_v], rows_v, sem).wait()

        def tok_body(t, tcarry):
            ttf = plsc.load_gather(
                tt_v, [jnp.full((16,), t, jnp.int32)]).astype(jnp.float32)
            acc = jnp.zeros((16,), jnp.float32)
            acc2 = jnp.zeros((16,), jnp.float32)
            for j in range(NV):
                sl = pl.ds(j * 16, 16)
                v = rows_v[t, sl] + (pos_v[t, sl] + ttf * tdiff_v[sl])
                rows_v[t, sl] = v
                acc = acc + v
                acc2 = acc2 + v * v
            mean = jnp.sum(acc) * (1.0 / H)
            var = jnp.sum(acc2) * (1.0 / H) - mean * mean + EPS
            seed_i = jnp.int32(0x5F3759DF) - lax.shift_right_logical(
                lax.bitcast_convert_type(var, jnp.int32), 1)
            y = lax.bitcast_convert_type(seed_i, jnp.float32)
            y = y * (1.5 - 0.5 * var * y * y)
            y = y * (1.5 - 0.5 * var * y * y)
            y = y * (1.5 - 0.5 * var * y * y)
            for j in range(NV):
                sl = pl.ds(j * 16, 16)
                nv = (rows_v[t, sl] - mean) * y
                rows_v[t, sl] = nv * gamma_v[sl] + beta_v[sl]
            return tcarry

        lax.fori_loop(0, C, tok_body, 0)
        pltpu.sync_copy(rows_v, out_hbm.at[pl.ds(base, C)])
        return carry

    lax.fori_loop(0, BG, chunk_body, 0)


@jax.jit
def _run(ids_flat, tt_flat, word_emb, pos_emb, type_emb, gamma, beta):
    mesh = plsc.VectorSubcoreMesh(core_axis_name="c", subcore_axis_name="s")
    f = pl.kernel(
        _body,
        out_type=jax.ShapeDtypeStruct((B * S, H), jnp.float32),
        mesh=mesh,
        scratch_types=[
            pltpu.VMEM((C,), jnp.int32),
            pltpu.VMEM((C,), jnp.int32),
            pltpu.VMEM((C, H), jnp.float32),
            pltpu.VMEM((C, H), jnp.float32),
            pltpu.VMEM((T, H), jnp.float32),
            pltpu.VMEM((H,), jnp.float32),
            pltpu.VMEM((H,), jnp.float32),
            pltpu.VMEM((H,), jnp.float32),
            pltpu.SemaphoreType.DMA,
        ],
        compiler_params=pltpu.CompilerParams(needs_layout_passes=False),
    )
    return f(ids_flat, tt_flat, word_emb, pos_emb, type_emb, gamma, beta)


def kernel(input_ids, token_type_ids, word_emb, pos_emb, type_emb, gamma,
           beta):
    ids_flat = input_ids.reshape(-1).astype(jnp.int32)
    tt_flat = token_type_ids.reshape(-1).astype(jnp.int32)
    out = _run(ids_flat, tt_flat, word_emb, pos_emb, type_emb, gamma, beta)
    return out.reshape(B, S, H)
